# Initial kernel scaffold; baseline (speedup 1.0000x reference)
import jax
import jax.numpy as jnp
from jax.experimental import pallas as pl


def kernel(u_in, w_num, cat_table):
    raise NotImplementedError("write your pallas kernel here")


# speedup vs baseline: 2.0489x; 2.0489x over previous
import functools

import jax
import jax.numpy as jnp
import numpy as np
from jax import lax
from jax.experimental import pallas as pl
from jax.experimental.pallas import tpu as pltpu
from jax.experimental.pallas import tpu_sc as plsc

_H = 64
_L = 64
_NT = 16
_NNT = 8
_CARD = 32
_N = 1024

_NC = 2
_NS = 16
_NW = _NC * _NS
_NPW = _N // _NW


def _pos_embed_np(seq_len, D=_H, max_len=10000.0):
    pos = np.arange(seq_len, dtype=np.float32)
    emb = np.zeros((seq_len, D), dtype=np.float32)
    for i in range(D // 2):
        omega_inv = max_len ** (2.0 * i / D)
        emb[:, 2 * i] = np.sin(pos / omega_inv)
        emb[:, 2 * i + 1] = np.cos(pos / omega_inv)
    return emb


_POS = _pos_embed_np(_L + 2)[:_L]

_BASE = np.empty((_L,), dtype=np.int32)
for _l in range(_L):
    _a, _m = _l // 16, _l % 16
    _BASE[_l] = (1024 + _a * 8 + _m) if _m < _NNT else (_a * 8 + (_m - _NNT)) * _CARD


def _body(t2_hbm, u_hbm, base_hbm, a_hbm, out_hbm,
          u_v, buf_v, idx_v, base_v, a_v, gsem):
    wid = lax.axis_index("s") * _NC + lax.axis_index("c")
    n0 = wid * _NPW

    pltpu.sync_copy(u_hbm.at[pl.ds(n0 * 192, _NPW * 192)], u_v)
    pltpu.sync_copy(base_hbm, base_v)
    pltpu.sync_copy(a_hbm, a_v)

    lane = lax.iota(jnp.int32, 16)
    cat_lane = lane >= _NNT

    def per_item(n, carry):
        uoff = n * 192
        for k in range(4):
            cls = plsc.load_gather(u_v, [uoff + 48 * k + 3 * lane])
            cls_i = (cls + 0.5).astype(jnp.int32)
            base_k = base_v[pl.ds(16 * k, 16)]
            idx_v[pl.ds(16 * k, 16)] = base_k + jnp.where(cat_lane, cls_i, 0)
        pltpu.async_copy(t2_hbm.at[idx_v], buf_v, gsem).wait()
        for a in range(4):
            for c in range(_NNT):
                l = 16 * a + c
                voff = jnp.full((16,), uoff + 3 * l + 1, dtype=jnp.int32)
                s = plsc.load_gather(u_v, [voff])
                for kk in range(4):
                    sl = pl.ds(16 * kk, 16)
                    buf_v[l, sl] = buf_v[l, sl] + s * a_v[c, sl]
        pltpu.sync_copy(buf_v, out_hbm.at[n0 + n])
        return carry

    lax.fori_loop(0, _NPW, per_item, 0)


def kernel(u_in, w_num, cat_table):
    pos = jnp.asarray(_POS)
    pos_cat = pos.reshape(4, 16, _H)[:, _NNT:, :]
    pos_num = pos.reshape(4, 16, _H)[:, :_NNT, :]
    w8 = w_num[: _NNT * _H].reshape(_NNT, _H)
    t_cat = (cat_table.reshape(_NNT, _CARD, _H)[None] +
             pos_cat[:, :, None, :]).reshape(1024, _H)
    t_num = (pos_num - w8[None]).reshape(32, _H)
    t2 = jnp.concatenate([t_cat, t_num], axis=0)
    a8 = 2.0 * w8
    u_flat = u_in.reshape(-1)
    base = jnp.asarray(_BASE)

    mesh = plsc.VectorSubcoreMesh(core_axis_name="c", subcore_axis_name="s",
                                  num_cores=_NC, num_subcores=_NS)
    run = pl.kernel(
        _body,
        out_type=jax.ShapeDtypeStruct((_N, _L, _H), jnp.float32),
        mesh=mesh,
        scratch_types=[
            pltpu.VMEM((_NPW * 192,), jnp.float32),
            pltpu.VMEM((_L, _H), jnp.float32),
            pltpu.VMEM((_L,), jnp.int32),
            pltpu.VMEM((_L,), jnp.int32),
            pltpu.VMEM((_NNT, _H), jnp.float32),
            pltpu.SemaphoreType.DMA,
        ],
        compiler_params=pltpu.CompilerParams(needs_layout_passes=False,
                                             use_tc_tiling_on_sc=False),
    )
    return run(t2, u_flat, base, a8)
